# baseline (speedup 1.0000x reference)
import functools

import jax
import jax.numpy as jnp
from jax import lax
from jax.experimental import pallas as pl
from jax.experimental.pallas import tpu as pltpu
from jax.experimental.pallas import tpu_sc as plsc

N = 4194304
B = 4096
ROWS = 8192
COLS = 1536
TRIPLES = COLS // 3
RB = 256

NC = 2
NS = 16
NW = NC * NS
CHUNK = N // NW
PIECE = 4096
NPIECE = CHUNK // PIECE
PROWS = PIECE // 128


def _tc_body(m_ref, s_ref, x_ref, ng_ref, sq_ref):
    x = x_ref[...]
    m = m_ref[...]
    d = x - m
    ng_ref[...] = -2.0 * d
    d2 = d * d
    sq_ref[...] = jax.lax.dot(d2, s_ref[...], precision=jax.lax.Precision.HIGHEST)


def _tc_call(posf, m_tiled, sel):
    grid = ROWS // RB
    return pl.pallas_call(
        _tc_body,
        grid=(grid,),
        in_specs=[
            pl.BlockSpec((1, COLS), lambda i: (0, 0)),
            pl.BlockSpec((COLS, TRIPLES), lambda i: (0, 0)),
            pl.BlockSpec((RB, COLS), lambda i: (i, 0)),
        ],
        out_specs=[
            pl.BlockSpec((RB, COLS), lambda i: (i, 0)),
            pl.BlockSpec((RB, TRIPLES), lambda i: (i, 0)),
        ],
        out_shape=[
            jax.ShapeDtypeStruct((ROWS, COLS), jnp.float32),
            jax.ShapeDtypeStruct((ROWS, TRIPLES), jnp.float32),
        ],
    )(m_tiled, sel, posf)


def _sc_seg_body(sq_hbm, ids_hbm, out_hbm, vals_v, ids_v, zero_v, accum_sh):
    cid = lax.axis_index("c")
    sid = lax.axis_index("s")
    wid = sid * NC + cid

    def _z(i, _):
        zero_v[pl.ds(i * 16, 16)] = jnp.zeros((16,), jnp.float32)
        return 0

    lax.fori_loop(0, B // 16, _z, 0)

    @pl.when(sid == 0)
    def _():
        pltpu.sync_copy(zero_v, accum_sh)

    plsc.subcore_barrier()

    base = wid * CHUNK

    def _piece(p, _):
        off = pl.multiple_of(base + p * PIECE, PIECE)
        pltpu.sync_copy(sq_hbm.at[pl.ds(off, PIECE)], vals_v)
        pltpu.sync_copy(ids_hbm.at[pl.ds(pl.multiple_of(off // 128, PROWS), PROWS)], ids_v)
        for j in range(PROWS):
            pltpu.sync_copy(
                vals_v.at[pl.ds(j * 128, 128)],
                accum_sh.at[ids_v.at[j]],
                add=True,
            )
        return 0

    lax.fori_loop(0, NPIECE, _piece, 0)

    plsc.subcore_barrier()

    @pl.when(sid == 0)
    def _():
        pltpu.sync_copy(accum_sh, out_hbm.at[cid])


def _sc_call(sq_flat, ids):
    mesh = plsc.VectorSubcoreMesh(core_axis_name="c", subcore_axis_name="s")
    f = functools.partial(
        pl.kernel,
        out_type=jax.ShapeDtypeStruct((NC, B), jnp.float32),
        mesh=mesh,
        scratch_types=[
            pltpu.VMEM((PIECE,), jnp.float32),
            pltpu.VMEM((PROWS, 128), jnp.int32),
            pltpu.VMEM((B,), jnp.float32),
            pltpu.VMEM_SHARED((B,), jnp.float32),
        ],
    )(_sc_seg_body)
    return f(sq_flat, ids.reshape(N // 128, 128))


def kernel(positions, segment_ids, minimum):
    posf = positions.reshape(ROWS, COLS)
    m_tiled = jnp.tile(minimum, TRIPLES).reshape(1, COLS)
    sel = (jnp.arange(COLS)[:, None] // 3
           == jnp.arange(TRIPLES)[None, :]).astype(jnp.float32)
    ng_flat, sq = _tc_call(posf, m_tiled, sel)
    neg_grad = ng_flat.reshape(N, 3)
    sq_flat = sq.reshape(N)
    partial = _sc_call(sq_flat, segment_ids.astype(jnp.int32))
    energies = partial[0] + partial[1]
    stress = jnp.zeros((B, 6), jnp.float32)
    return (energies, neg_grad, stress)


# speedup vs baseline: 14.4158x; 14.4158x over previous
import functools

import jax
import jax.numpy as jnp
from jax import lax
from jax.experimental import pallas as pl
from jax.experimental.pallas import tpu as pltpu
from jax.experimental.pallas import tpu_sc as plsc

N = 4194304
B = 4096

NC = 2
NS = 16
NW = NC * NS
CHUNK = N // NW
PIECE = 4096
NPIECE = CHUNK // PIECE
PROWS = PIECE // 128


def _sc_seg_body(sq_hbm, ids_hbm, out_hbm, vals_v, ids_v, zero_v, accum_sh):
    cid = lax.axis_index("c")
    sid = lax.axis_index("s")
    wid = sid * NC + cid

    def _z(i, _):
        zero_v[pl.ds(i * 16, 16)] = jnp.zeros((16,), jnp.float32)
        return 0

    lax.fori_loop(0, B // 16, _z, 0)

    @pl.when(sid == 0)
    def _():
        pltpu.sync_copy(zero_v, accum_sh)

    plsc.subcore_barrier()

    base_row = wid * (CHUNK // 128)

    def _piece(p, _):
        row = pl.multiple_of(base_row + p * PROWS, PROWS)
        pltpu.sync_copy(sq_hbm.at[pl.ds(row, PROWS)], vals_v)
        pltpu.sync_copy(ids_hbm.at[pl.ds(row, PROWS)], ids_v)
        for j in range(PROWS):
            pltpu.sync_copy(
                vals_v.at[j],
                accum_sh.at[ids_v.at[j]],
                add=True,
            )
        return 0

    lax.fori_loop(0, NPIECE, _piece, 0)

    plsc.subcore_barrier()

    @pl.when(sid == 0)
    def _():
        pltpu.sync_copy(accum_sh, out_hbm.at[cid])


def _sc_call(sq2d, ids2d):
    mesh = plsc.VectorSubcoreMesh(core_axis_name="c", subcore_axis_name="s")
    f = functools.partial(
        pl.kernel,
        out_type=jax.ShapeDtypeStruct((NC, B), jnp.float32),
        mesh=mesh,
        scratch_types=[
            pltpu.VMEM((PROWS, 128), jnp.float32),
            pltpu.VMEM((PROWS, 128), jnp.int32),
            pltpu.VMEM((B,), jnp.float32),
            pltpu.VMEM_SHARED((B,), jnp.float32),
        ],
    )(_sc_seg_body)
    return f(sq2d, ids2d)


def kernel(positions, segment_ids, minimum):
    d = positions - minimum
    neg_grad = -2.0 * d
    sq = jnp.sum(d * d, axis=1)
    partial = _sc_call(sq.reshape(N // 128, 128),
                       segment_ids.astype(jnp.int32).reshape(N // 128, 128))
    energies = partial[0] + partial[1]
    stress = jnp.zeros((B, 6), jnp.float32)
    return (energies, neg_grad, stress)
